# baseline (speedup 1.0000x reference)
import functools

import jax
import jax.numpy as jnp
from jax import lax
from jax.experimental import pallas as pl
from jax.experimental.pallas import tpu as pltpu
from jax.experimental.pallas import tpu_sc as plsc

_NC = 2
_NS = 16
_NW = _NC * _NS
_EP = 128
_G = 4096
_TB1 = 2048
_TB2 = 2048
_HI = 32
_LO = 32
_C8 = 8


def _make_sc_gather(G):
    t_pw = G // _NW
    n_tr = -(-t_pw // 128)
    assert t_pw % n_tr == 0
    per = t_pw // n_tr
    mesh = plsc.VectorSubcoreMesh(
        core_axis_name="c", subcore_axis_name="s",
        num_cores=_NC, num_subcores=_NS,
    )

    @functools.partial(
        pl.kernel,
        out_type=jax.ShapeDtypeStruct((G, _EP), jnp.float32),
        mesh=mesh,
        scratch_types=[
            pltpu.VMEM((t_pw,), jnp.int32),
            pltpu.VMEM((t_pw, _EP), jnp.float32),
            pltpu.SemaphoreType.DMA,
        ],
    )
    def gather(ids_hbm, emb_hbm, out_hbm, idx_v, rows_v, sem):
        wid = lax.axis_index("s") * _NC + lax.axis_index("c")
        base = wid * t_pw
        pltpu.sync_copy(ids_hbm.at[pl.ds(base, t_pw)], idx_v)
        handles = [
            pltpu.async_copy(
                emb_hbm.at[idx_v.at[pl.ds(c * per, per)]],
                rows_v.at[pl.ds(c * per, per)],
                sem,
            )
            for c in range(n_tr)
        ]
        for h in handles:
            h.wait()
        pltpu.sync_copy(rows_v, out_hbm.at[pl.ds(base, t_pw)])

    return gather


def _onehot_body(TB, ids_ref, emb_ref, w_ref, o_ref):
    ids = ids_ref[...]
    lo16 = (ids & (_LO - 1)).astype(jnp.int16)
    hi = ids >> 5
    iota_lo = lax.broadcasted_iota(jnp.int16, (_LO, TB), 0)
    iota_hi = lax.broadcasted_iota(jnp.int32, (_HI, TB), 0)
    oh_lo = jnp.where(iota_lo == lax.broadcast_in_dim(lo16, (_LO, TB), (1,)),
                      jnp.bfloat16(1.0), jnp.bfloat16(0.0))
    oh_hi = jnp.where(iota_hi == lax.broadcast_in_dim(hi, (_HI, TB), (1,)),
                      jnp.float32(1.0), jnp.float32(0.0))
    emb3r = jnp.concatenate(
        [emb_ref[pl.ds(a * _LO, _LO), pl.ds(0, _C8)] for a in range(_HI)],
        axis=1).astype(jnp.bfloat16)
    g8 = lax.dot_general(
        emb3r, oh_lo,
        dimension_numbers=(((0,), (0,)), ((), ())),
        preferred_element_type=jnp.float32,
    )
    h8 = g8[0:_C8, :] * oh_hi[0:1, :]
    for a in range(1, _HI):
        h8 = h8 + g8[a * _C8:(a + 1) * _C8, :] * oh_hi[a:a + 1, :]
    o_ref[...] = lax.dot_general(
        w_ref[...], h8.astype(jnp.bfloat16),
        dimension_numbers=(((1,), (0,)), ((), ())),
        preferred_element_type=jnp.float32,
    )


def _tc1_onehot_proj(ids3, emb_pad_bf, w8_t, V, T, G):
    nb = (T - G) // _TB1
    off = G // _TB1
    return pl.pallas_call(
        functools.partial(_onehot_body, _TB1),
        grid=(nb,),
        in_specs=[
            pl.BlockSpec((_TB1,), lambda i: (i + off,)),
            pl.BlockSpec((_HI * _LO, _EP), lambda i: (0, 0)),
            pl.BlockSpec((V, _C8), lambda i: (0, 0)),
        ],
        out_specs=pl.BlockSpec((V, _TB1), lambda i: (0, i + off)),
        out_shape=jax.ShapeDtypeStruct((V, T), jnp.float32),
    )(ids3, emb_pad_bf, w8_t)


_RS = 200


def _tc2_body(prev_ref, w_ref, h_ref, o_ref, h8_s):
    @pl.when(pl.program_id(1) == 0)
    def _():
        h8_s[...] = h_ref[...][:, :_C8].astype(jnp.bfloat16)
    o_ref[...] = lax.dot_general(
        w_ref[...], h8_s[...],
        dimension_numbers=(((1,), (1,)), ((), ())),
        preferred_element_type=jnp.float32,
    )


def _tc2_fill(prev, w8_t, hidden_lo, V, T, G):
    return pl.pallas_call(
        _tc2_body,
        grid=(G // _TB2, V // _RS),
        in_specs=[
            pl.BlockSpec(memory_space=pl.ANY),
            pl.BlockSpec((_RS, _C8), lambda i, j: (j, 0)),
            pl.BlockSpec((_TB2, _EP), lambda i, j: (i, 0)),
        ],
        out_specs=pl.BlockSpec((_RS, _TB2), lambda i, j: (j, i)),
        out_shape=jax.ShapeDtypeStruct((V, T), jnp.float32),
        input_output_aliases={0: 0},
        scratch_shapes=[pltpu.VMEM((_TB2, _C8), jnp.bfloat16)],
    )(prev, w8_t, hidden_lo)


def kernel(input_ids, cu_seq_lens_q, cu_seq_lens_k, max_length_q,
           max_length_k, position_ids, text_position_ids, pack_num_samples,
           embed_table, W, b):
    B, T0 = input_ids.shape
    V, D = embed_table.shape
    T = B * T0
    ids = input_ids.reshape(-1).astype(jnp.int32)
    ids3 = ids
    emb_pad = jnp.pad(
        jnp.concatenate(
            [embed_table, jnp.ones((V, 1), jnp.float32),
             jnp.zeros((V, _EP - D - 1), jnp.float32)], axis=1),
        ((0, _HI * _LO - V), (0, 0)))
    w8_t = jnp.concatenate(
        [W, b.reshape(V, 1), jnp.zeros((V, _C8 - D - 1), jnp.float32)],
        axis=1).astype(jnp.bfloat16)
    hidden_lo = _make_sc_gather(_G)(ids, emb_pad)
    out_t = _tc1_onehot_proj(ids3, emb_pad, w8_t, V, T, _G)
    out_t = _tc2_fill(out_t, w8_t, hidden_lo, V, T, _G)
    return jnp.transpose(out_t).reshape(B, T0, V)


# speedup vs baseline: 1.0734x; 1.0734x over previous
import functools

import jax
import jax.numpy as jnp
from jax import lax
from jax.experimental import pallas as pl
from jax.experimental.pallas import tpu as pltpu
from jax.experimental.pallas import tpu_sc as plsc

_NC = 2
_NS = 16
_NW = _NC * _NS
_EP = 128
_G = 4096
_TB1 = 2048
_TB2 = 2048
_HI = 32
_LO = 32
_C8 = 8


def _make_sc_gather(G):
    t_pw = G // _NW
    n_tr = -(-t_pw // 128)
    assert t_pw % n_tr == 0
    per = t_pw // n_tr
    mesh = plsc.VectorSubcoreMesh(
        core_axis_name="c", subcore_axis_name="s",
        num_cores=_NC, num_subcores=_NS,
    )

    @functools.partial(
        pl.kernel,
        out_type=jax.ShapeDtypeStruct((G, _EP), jnp.float32),
        mesh=mesh,
        scratch_types=[
            pltpu.VMEM((t_pw,), jnp.int32),
            pltpu.VMEM((t_pw, _EP), jnp.float32),
            pltpu.SemaphoreType.DMA,
        ],
    )
    def gather(ids_hbm, emb_hbm, out_hbm, idx_v, rows_v, sem):
        wid = lax.axis_index("s") * _NC + lax.axis_index("c")
        base = wid * t_pw
        pltpu.sync_copy(ids_hbm.at[pl.ds(base, t_pw)], idx_v)
        handles = [
            pltpu.async_copy(
                emb_hbm.at[idx_v.at[pl.ds(c * per, per)]],
                rows_v.at[pl.ds(c * per, per)],
                sem,
            )
            for c in range(n_tr)
        ]
        for h in handles:
            h.wait()
        pltpu.sync_copy(rows_v, out_hbm.at[pl.ds(base, t_pw)])

    return gather


def _onehot_body(TB, ids_ref, emb_ref, w_ref, o_ref):
    ids = ids_ref[...]
    lo16 = (ids & (_LO - 1)).astype(jnp.int16)
    hi = ids >> 5
    iota_lo = lax.broadcasted_iota(jnp.int16, (_LO, TB), 0)
    iota_hi = lax.broadcasted_iota(jnp.int32, (_HI, TB), 0)
    oh_lo = jnp.where(iota_lo == lax.broadcast_in_dim(lo16, (_LO, TB), (1,)),
                      jnp.bfloat16(1.0), jnp.bfloat16(0.0))
    oh_hi = jnp.where(iota_hi == lax.broadcast_in_dim(hi, (_HI, TB), (1,)),
                      jnp.float32(1.0), jnp.float32(0.0))
    emb3r = jnp.concatenate(
        [emb_ref[pl.ds(a * _LO, _LO), pl.ds(0, _C8)] for a in range(_HI)],
        axis=1).astype(jnp.bfloat16)
    g8 = lax.dot_general(
        emb3r, oh_lo,
        dimension_numbers=(((0,), (0,)), ((), ())),
        preferred_element_type=jnp.float32,
    )
    h8 = g8[0:_C8, :] * oh_hi[0:1, :]
    for a in range(1, _HI):
        h8 = h8 + g8[a * _C8:(a + 1) * _C8, :] * oh_hi[a:a + 1, :]
    o_ref[...] = lax.dot_general(
        w_ref[...], h8.astype(jnp.bfloat16),
        dimension_numbers=(((1,), (0,)), ((), ())),
        preferred_element_type=jnp.float32,
    )


def _tc1_onehot_proj(ids3, emb_pad_bf, w8_t, V, T, G):
    nb = (T - G) // _TB1
    off = G // _TB1
    return pl.pallas_call(
        functools.partial(_onehot_body, _TB1),
        grid=(nb,),
        in_specs=[
            pl.BlockSpec((_TB1,), lambda i: (i + off,)),
            pl.BlockSpec((_HI * _LO, _EP), lambda i: (0, 0)),
            pl.BlockSpec((V, _C8), lambda i: (0, 0)),
        ],
        out_specs=pl.BlockSpec((V, _TB1), lambda i: (0, i + off)),
        out_shape=jax.ShapeDtypeStruct((V, T), jnp.float32),
    )(ids3, emb_pad_bf, w8_t)


def _tc2_body(prev_ref, w_ref, h_ref, o_ref):
    h8 = h_ref[...][:, :_C8].astype(jnp.bfloat16)
    o_ref[...] = lax.dot_general(
        w_ref[...], h8,
        dimension_numbers=(((1,), (1,)), ((), ())),
        preferred_element_type=jnp.float32,
    )


def _tc2_fill(prev, w8_t, hidden_lo, V, T, G):
    return pl.pallas_call(
        _tc2_body,
        grid=(G // _TB2,),
        in_specs=[
            pl.BlockSpec(memory_space=pl.ANY),
            pl.BlockSpec((V, _C8), lambda i: (0, 0)),
            pl.BlockSpec((_TB2, _EP), lambda i: (i, 0)),
        ],
        out_specs=pl.BlockSpec((V, _TB2), lambda i: (0, i)),
        out_shape=jax.ShapeDtypeStruct((V, T), jnp.float32),
        input_output_aliases={0: 0},
    )(prev, w8_t, hidden_lo)


def kernel(input_ids, cu_seq_lens_q, cu_seq_lens_k, max_length_q,
           max_length_k, position_ids, text_position_ids, pack_num_samples,
           embed_table, W, b):
    B, T0 = input_ids.shape
    V, D = embed_table.shape
    T = B * T0
    ids = input_ids.reshape(-1).astype(jnp.int32)
    ids3 = ids
    emb_pad = jnp.pad(
        jnp.concatenate(
            [embed_table, jnp.ones((V, 1), jnp.float32),
             jnp.zeros((V, _EP - D - 1), jnp.float32)], axis=1),
        ((0, _HI * _LO - V), (0, 0)))
    w8_t = jnp.concatenate(
        [W, b.reshape(V, 1), jnp.zeros((V, _C8 - D - 1), jnp.float32)],
        axis=1).astype(jnp.bfloat16)
    hidden_lo = _make_sc_gather(_G)(ids, emb_pad)
    out_t = _tc1_onehot_proj(ids3, emb_pad, w8_t, V, T, _G)
    out_t = _tc2_fill(out_t, w8_t, hidden_lo, V, T, _G)
    return jnp.transpose(out_t).reshape(B, T0, V)


# speedup vs baseline: 1.1782x; 1.0977x over previous
import functools

import jax
import jax.numpy as jnp
from jax import lax
from jax.experimental import pallas as pl
from jax.experimental.pallas import tpu as pltpu
from jax.experimental.pallas import tpu_sc as plsc

_NC = 2
_NS = 16
_NW = _NC * _NS
_EP = 128
_G = 4096
_TB1 = 2048
_TB2 = 2048
_HI = 32
_LO = 32
_C8 = 8


def _make_sc_gather(G):
    t_pw = G // _NW
    n_tr = -(-t_pw // 128)
    assert t_pw % n_tr == 0
    per = t_pw // n_tr
    mesh = plsc.VectorSubcoreMesh(
        core_axis_name="c", subcore_axis_name="s",
        num_cores=_NC, num_subcores=_NS,
    )

    @functools.partial(
        pl.kernel,
        out_type=jax.ShapeDtypeStruct((G, _EP), jnp.float32),
        mesh=mesh,
        scratch_types=[
            pltpu.VMEM((t_pw,), jnp.int32),
            pltpu.VMEM((t_pw, _EP), jnp.float32),
            pltpu.SemaphoreType.DMA,
        ],
    )
    def gather(ids_hbm, emb_hbm, out_hbm, idx_v, rows_v, sem):
        wid = lax.axis_index("s") * _NC + lax.axis_index("c")
        base = wid * t_pw
        pltpu.sync_copy(ids_hbm.at[pl.ds(base, t_pw)], idx_v)
        handles = [
            pltpu.async_copy(
                emb_hbm.at[idx_v.at[pl.ds(c * per, per)]],
                rows_v.at[pl.ds(c * per, per)],
                sem,
            )
            for c in range(n_tr)
        ]
        for h in handles:
            h.wait()
        pltpu.sync_copy(rows_v, out_hbm.at[pl.ds(base, t_pw)])

    return gather


def _w8(w_ref, b_ref):
    return jnp.concatenate(
        [w_ref[...], b_ref[...],
         jnp.zeros((_C8 - w_ref.shape[0] - 1, w_ref.shape[1]), jnp.float32)],
        axis=0).astype(jnp.bfloat16)


def _onehot_body(TB, ids_ref, emb_ref, w_ref, b_ref, o_ref):
    ids = ids_ref[...]
    lo16 = (ids & (_LO - 1)).astype(jnp.int16)
    hi = ids >> 5
    iota_lo = lax.broadcasted_iota(jnp.int16, (_LO, TB), 0)
    iota_hi = lax.broadcasted_iota(jnp.int32, (_HI, TB), 0)
    oh_lo = jnp.where(iota_lo == lax.broadcast_in_dim(lo16, (_LO, TB), (1,)),
                      jnp.bfloat16(1.0), jnp.bfloat16(0.0))
    oh_hi = jnp.where(iota_hi == lax.broadcast_in_dim(hi, (_HI, TB), (1,)),
                      jnp.float32(1.0), jnp.float32(0.0))
    emb3r = jnp.concatenate(
        [emb_ref[pl.ds(a * _LO, _LO), pl.ds(0, _C8)] for a in range(_HI)],
        axis=1).astype(jnp.bfloat16)
    g8 = lax.dot_general(
        emb3r, oh_lo,
        dimension_numbers=(((0,), (0,)), ((), ())),
        preferred_element_type=jnp.float32,
    )
    h8 = g8[0:_C8, :] * oh_hi[0:1, :]
    for a in range(1, _HI):
        h8 = h8 + g8[a * _C8:(a + 1) * _C8, :] * oh_hi[a:a + 1, :]
    o_ref[...] = lax.dot_general(
        _w8(w_ref, b_ref), h8.astype(jnp.bfloat16),
        dimension_numbers=(((0,), (0,)), ((), ())),
        preferred_element_type=jnp.float32,
    )


def _tc1_onehot_proj(ids3, emb_pad_bf, w_t, b_row, V, T, G):
    nb = (T - G) // _TB1
    off = G // _TB1
    D = w_t.shape[0]
    return pl.pallas_call(
        functools.partial(_onehot_body, _TB1),
        grid=(nb,),
        in_specs=[
            pl.BlockSpec((_TB1,), lambda i: (i + off,)),
            pl.BlockSpec((_HI * _LO, _EP), lambda i: (0, 0)),
            pl.BlockSpec((D, V), lambda i: (0, 0)),
            pl.BlockSpec((1, V), lambda i: (0, 0)),
        ],
        out_specs=pl.BlockSpec((V, _TB1), lambda i: (0, i + off)),
        out_shape=jax.ShapeDtypeStruct((V, T), jnp.float32),
    )(ids3, emb_pad_bf, w_t, b_row)


def _tc2_body(prev_ref, w_ref, b_ref, h_ref, o_ref):
    h8 = h_ref[...][:, :_C8].astype(jnp.bfloat16)
    o_ref[...] = lax.dot_general(
        _w8(w_ref, b_ref), h8,
        dimension_numbers=(((0,), (1,)), ((), ())),
        preferred_element_type=jnp.float32,
    )


def _tc2_fill(prev, w_t, b_row, hidden_lo, V, T, G):
    D = w_t.shape[0]
    return pl.pallas_call(
        _tc2_body,
        grid=(G // _TB2,),
        in_specs=[
            pl.BlockSpec(memory_space=pl.ANY),
            pl.BlockSpec((D, V), lambda i: (0, 0)),
            pl.BlockSpec((1, V), lambda i: (0, 0)),
            pl.BlockSpec((_TB2, _EP), lambda i: (i, 0)),
        ],
        out_specs=pl.BlockSpec((V, _TB2), lambda i: (0, i)),
        out_shape=jax.ShapeDtypeStruct((V, T), jnp.float32),
        input_output_aliases={0: 0},
    )(prev, w_t, b_row, hidden_lo)


def kernel(input_ids, cu_seq_lens_q, cu_seq_lens_k, max_length_q,
           max_length_k, position_ids, text_position_ids, pack_num_samples,
           embed_table, W, b):
    B, T0 = input_ids.shape
    V, D = embed_table.shape
    T = B * T0
    ids = input_ids.reshape(-1).astype(jnp.int32)
    ids3 = ids
    emb_pad = jnp.pad(
        jnp.concatenate(
            [embed_table, jnp.ones((V, 1), jnp.float32),
             jnp.zeros((V, _EP - D - 1), jnp.float32)], axis=1),
        ((0, _HI * _LO - V), (0, 0)))
    w_t = W.T
    b_row = b.reshape(1, V)
    hidden_lo = _make_sc_gather(_G)(ids, emb_pad)
    out_t = _tc1_onehot_proj(ids3, emb_pad, w_t, b_row, V, T, _G)
    out_t = _tc2_fill(out_t, w_t, b_row, hidden_lo, V, T, _G)
    return jnp.transpose(out_t).reshape(B, T0, V)


# speedup vs baseline: 1.1817x; 1.0030x over previous
import functools

import jax
import jax.numpy as jnp
from jax import lax
from jax.experimental import pallas as pl
from jax.experimental.pallas import tpu as pltpu
from jax.experimental.pallas import tpu_sc as plsc

_NC = 2
_NS = 16
_NW = _NC * _NS
_EP = 128
_G = 4096
_TB1 = 2048
_TB2 = 2048
_HI = 32
_LO = 32
_C8 = 8


def _make_sc_gather(G):
    t_pw = G // _NW
    n_tr = -(-t_pw // 128)
    assert t_pw % n_tr == 0
    per = t_pw // n_tr
    mesh = plsc.VectorSubcoreMesh(
        core_axis_name="c", subcore_axis_name="s",
        num_cores=_NC, num_subcores=_NS,
    )

    @functools.partial(
        pl.kernel,
        out_type=jax.ShapeDtypeStruct((G, _EP), jnp.float32),
        mesh=mesh,
        scratch_types=[
            pltpu.VMEM((t_pw,), jnp.int32),
            pltpu.VMEM((t_pw, _EP), jnp.float32),
            pltpu.SemaphoreType.DMA,
        ],
    )
    def gather(ids_hbm, emb_hbm, out_hbm, idx_v, rows_v, sem):
        wid = lax.axis_index("s") * _NC + lax.axis_index("c")
        base = wid * t_pw
        pltpu.sync_copy(ids_hbm.at[0, pl.ds(base, t_pw)], idx_v)
        handles = [
            pltpu.async_copy(
                emb_hbm.at[idx_v.at[pl.ds(c * per, per)]],
                rows_v.at[pl.ds(c * per, per)],
                sem,
            )
            for c in range(n_tr)
        ]
        for h in handles:
            h.wait()
        pltpu.sync_copy(rows_v, out_hbm.at[pl.ds(base, t_pw)])

    return gather


def _w8(w_ref, b_ref):
    return jnp.concatenate(
        [w_ref[...], b_ref[...],
         jnp.zeros((_C8 - w_ref.shape[0] - 1, w_ref.shape[1]), jnp.float32)],
        axis=0).astype(jnp.bfloat16)


def _onehot_body(TB, ids_ref, emb_ref, w_ref, b_ref, o_ref):
    ids = ids_ref[0, :]
    lo16 = (ids & (_LO - 1)).astype(jnp.int16)
    hi = ids >> 5
    iota_lo = lax.broadcasted_iota(jnp.int16, (_LO, TB), 0)
    iota_hi = lax.broadcasted_iota(jnp.int32, (_HI, TB), 0)
    oh_lo = jnp.where(iota_lo == lax.broadcast_in_dim(lo16, (_LO, TB), (1,)),
                      jnp.bfloat16(1.0), jnp.bfloat16(0.0))
    oh_hi = jnp.where(iota_hi == lax.broadcast_in_dim(hi, (_HI, TB), (1,)),
                      jnp.float32(1.0), jnp.float32(0.0))
    emb3r = jnp.concatenate(
        [emb_ref[pl.ds(a * _LO, _LO), pl.ds(0, _C8)] for a in range(_HI)],
        axis=1).astype(jnp.bfloat16)
    g8 = lax.dot_general(
        emb3r, oh_lo,
        dimension_numbers=(((0,), (0,)), ((), ())),
        preferred_element_type=jnp.float32,
    )
    h8 = g8[0:_C8, :] * oh_hi[0:1, :]
    for a in range(1, _HI):
        h8 = h8 + g8[a * _C8:(a + 1) * _C8, :] * oh_hi[a:a + 1, :]
    o_ref[...] = lax.dot_general(
        _w8(w_ref, b_ref), h8.astype(jnp.bfloat16),
        dimension_numbers=(((0,), (0,)), ((), ())),
        preferred_element_type=jnp.float32,
    )


def _tc1_onehot_proj(ids3, emb_pad_bf, w_t, b_row, V, T, G):
    nb = (T - G) // _TB1
    off = G // _TB1
    D = w_t.shape[0]
    return pl.pallas_call(
        functools.partial(_onehot_body, _TB1),
        grid=(nb,),
        in_specs=[
            pl.BlockSpec((1, _TB1), lambda i: (0, i + off)),
            pl.BlockSpec((_HI * _LO, _EP), lambda i: (0, 0)),
            pl.BlockSpec((D, V), lambda i: (0, 0)),
            pl.BlockSpec((1, V), lambda i: (0, 0)),
        ],
        out_specs=pl.BlockSpec((V, _TB1), lambda i: (0, i + off)),
        out_shape=jax.ShapeDtypeStruct((V, T), jnp.float32),
    )(ids3, emb_pad_bf, w_t, b_row)


def _tc2_body(prev_ref, w_ref, b_ref, h_ref, o_ref):
    h8 = h_ref[...][:, :_C8].astype(jnp.bfloat16)
    o_ref[...] = lax.dot_general(
        _w8(w_ref, b_ref), h8,
        dimension_numbers=(((0,), (1,)), ((), ())),
        preferred_element_type=jnp.float32,
    )


def _tc2_fill(prev, w_t, b_row, hidden_lo, V, T, G):
    D = w_t.shape[0]
    return pl.pallas_call(
        _tc2_body,
        grid=(G // _TB2,),
        in_specs=[
            pl.BlockSpec(memory_space=pl.ANY),
            pl.BlockSpec((D, V), lambda i: (0, 0)),
            pl.BlockSpec((1, V), lambda i: (0, 0)),
            pl.BlockSpec((_TB2, _EP), lambda i: (i, 0)),
        ],
        out_specs=pl.BlockSpec((V, _TB2), lambda i: (0, i)),
        out_shape=jax.ShapeDtypeStruct((V, T), jnp.float32),
        input_output_aliases={0: 0},
    )(prev, w_t, b_row, hidden_lo)


def kernel(input_ids, cu_seq_lens_q, cu_seq_lens_k, max_length_q,
           max_length_k, position_ids, text_position_ids, pack_num_samples,
           embed_table, W, b):
    B, T0 = input_ids.shape
    V, D = embed_table.shape
    T = B * T0
    ids = input_ids.astype(jnp.int32)
    emb_pad = jnp.pad(
        jnp.concatenate(
            [embed_table, jnp.ones((V, 1), jnp.float32),
             jnp.zeros((V, _EP - D - 1), jnp.float32)], axis=1),
        ((0, _HI * _LO - V), (0, 0)))
    w_t = W.T
    b_row = b.reshape(1, V)
    hidden_lo = _make_sc_gather(_G)(ids, emb_pad)
    out_t = _tc1_onehot_proj(ids, emb_pad, w_t, b_row, V, T, _G)
    out_t = _tc2_fill(out_t, w_t, b_row, hidden_lo, V, T, _G)
    return jnp.transpose(out_t).reshape(B, T0, V)
